# baseline (speedup 1.0000x reference)
import functools

import jax
import jax.numpy as jnp
from jax import lax
from jax.experimental import pallas as pl
from jax.experimental.pallas import tpu as pltpu
from jax.experimental.pallas import tpu_sc as plsc

N = 10000
E = 320000
F_IN = 128
HID = 64
HEADS = 8
NCLS = 40
SLOPE = 0.2

W1 = 80
W2 = 48
NC = 2
NS = 16
BN = 1000
SPAN = 640
TAIL = N - 15 * SPAN

NBLK = N // BN
C1 = 400
ET1 = E // NS
C2 = 400
ET2 = E // (NC * NS)

_MESH = plsc.VectorSubcoreMesh(core_axis_name="c", subcore_axis_name="s")


def _elu(v):
    return jnp.where(v > 0, v, jnp.exp(jnp.minimum(v, 0.0)) - 1.0)


def _dense1_body(x_ref, wh_ref, ah_ref, h_ref, f1_ref, f2_ref):
    xb = x_ref[...]
    for h in range(HEADS):
        hm = jnp.dot(xb, wh_ref[h], preferred_element_type=jnp.float32)
        h_ref[h, :, 0:HID] = hm
        h_ref[h, :, HID:HID + 1] = jnp.ones((BN, 1), jnp.float32)
        h_ref[h, :, HID + 1:W1] = jnp.zeros((BN, W1 - HID - 1), jnp.float32)
        f1_ref[0, h, :] = jnp.dot(hm, ah_ref[h, 0:HID])
        f2_ref[0, h, :] = jnp.dot(hm, ah_ref[h, HID:2 * HID])


def _dense1(x, Wh, ah):
    return pl.pallas_call(
        _dense1_body,
        grid=(N // BN,),
        in_specs=[
            pl.BlockSpec((BN, F_IN), lambda i: (i, 0)),
            pl.BlockSpec((HEADS, F_IN, HID), lambda i: (0, 0, 0)),
            pl.BlockSpec((HEADS, 2 * HID), lambda i: (0, 0)),
        ],
        out_specs=[
            pl.BlockSpec((HEADS, BN, W1), lambda i: (0, i, 0)),
            pl.BlockSpec((1, HEADS, BN), lambda i: (i, 0, 0)),
            pl.BlockSpec((1, HEADS, BN), lambda i: (i, 0, 0)),
        ],
        out_shape=[
            jax.ShapeDtypeStruct((HEADS, N, W1), jnp.float32),
            jax.ShapeDtypeStruct((NBLK, HEADS, BN), jnp.float32),
            jax.ShapeDtypeStruct((NBLK, HEADS, BN), jnp.float32),
        ],
    )(x, Wh, ah)


def _edges1_body(h_hbm, f1_hbm, f2_hbm, src_hbm, dst_hbm, out_hbm,
                 f1_vm, f2_vm, srcb, dstb, didx, rows, wbuf, agg_sh):
    c = lax.axis_index("c")
    s = lax.axis_index("s")
    ebase = s * ET1
    rbase = s * SPAN

    def zero_rows(_):
        def zr(e, carry):
            for k in range(W1 // 16):
                rows[e, pl.ds(k * 16, 16)] = jnp.zeros((16,), jnp.float32)
            return carry
        lax.fori_loop(0, C1, zr, 0)

    def per_head(i, carry):
        h = c * (HEADS // NC) + i
        for k in range(NBLK):
            pltpu.sync_copy(f1_hbm.at[pl.ds((k * HEADS + h) * BN, BN)],
                            f1_vm.at[pl.ds(k * BN, BN)])
            pltpu.sync_copy(f2_hbm.at[pl.ds((k * HEADS + h) * BN, BN)],
                            f2_vm.at[pl.ds(k * BN, BN)])
        zero_rows(None)

        @pl.when(s < NS - 1)
        def _():
            pltpu.sync_copy(rows, agg_sh.at[pl.ds(rbase, C1)])
            pltpu.sync_copy(rows.at[pl.ds(0, SPAN - C1)],
                            agg_sh.at[pl.ds(rbase + C1, SPAN - C1)])

        @pl.when(s == NS - 1)
        def _():
            pltpu.sync_copy(rows, agg_sh.at[pl.ds(15 * SPAN, TAIL)])
        plsc.subcore_barrier()

        def per_chunk(k, carry2):
            e0 = ebase + k * C1
            pltpu.sync_copy(src_hbm.at[pl.ds(e0, C1)], srcb)
            pltpu.sync_copy(dst_hbm.at[pl.ds(e0, C1)], dstb)

            def wloop(j, carry3):
                sv = srcb[pl.ds(j * 16, 16)]
                dv = dstb[pl.ds(j * 16, 16)]
                z = (plsc.load_gather(f1_vm, [sv])
                     + plsc.load_gather(f2_vm, [dv]))
                wbuf[pl.ds(j * 16, 16)] = jnp.exp(
                    jnp.minimum(-z, -SLOPE * z))
                didx[pl.ds(j * 16, 16)] = dv + h * N
                return carry3
            lax.fori_loop(0, C1 // 16, wloop, 0)

            pltpu.sync_copy(h_hbm.at[didx], rows)

            def eloop(e, carry3):
                wv = plsc.load_gather(wbuf, [jnp.full((16,), e, jnp.int32)])
                for k2 in range(W1 // 16):
                    rows[e, pl.ds(k2 * 16, 16)] = (
                        rows[e, pl.ds(k2 * 16, 16)] * wv)
                return carry3
            lax.fori_loop(0, C1, eloop, 0)

            pltpu.sync_copy(rows, agg_sh.at[srcb], add=True)
            return carry2
        lax.fori_loop(0, ET1 // C1, per_chunk, 0)
        plsc.subcore_barrier()

        @pl.when(s < NS - 1)
        def _():
            pltpu.sync_copy(agg_sh.at[pl.ds(rbase, SPAN)],
                            out_hbm.at[pl.ds(h * N + rbase, SPAN)])

        @pl.when(s == NS - 1)
        def _():
            pltpu.sync_copy(agg_sh.at[pl.ds(15 * SPAN, TAIL)],
                            out_hbm.at[pl.ds(h * N + 15 * SPAN, TAIL)])
        plsc.subcore_barrier()
        return carry
    lax.fori_loop(0, HEADS // NC, per_head, 0)


def _edges1(h_flat, f1_flat, f2_flat, src, dst):
    return pl.kernel(
        _edges1_body,
        mesh=_MESH,
        compiler_params=pltpu.CompilerParams(needs_layout_passes=False, use_tc_tiling_on_sc=False),
        out_type=jax.ShapeDtypeStruct((HEADS * N, W1), jnp.float32),
        scratch_types=[
            pltpu.VMEM((N,), jnp.float32),
            pltpu.VMEM((N,), jnp.float32),
            pltpu.VMEM((C1,), jnp.int32),
            pltpu.VMEM((C1,), jnp.int32),
            pltpu.VMEM((C1,), jnp.int32),
            pltpu.VMEM((C1, W1), jnp.float32),
            pltpu.VMEM((C1,), jnp.float32),
            pltpu.VMEM_SHARED((N, W1), jnp.float32),
        ],
    )(h_flat, f1_flat, f2_flat, src, dst)


def _dense2_body(agg_ref, wo_ref, ao_ref, h2_ref, fo_ref):
    parts = []
    for h in range(HEADS):
        a = agg_ref[h, :, 0:HID]
        rs = agg_ref[h, :, HID:HID + 1]
        parts.append(_elu(a / (rs + 1e-16)))
    h1 = jnp.concatenate(parts, axis=1)
    h2 = jnp.dot(h1, wo_ref[...], preferred_element_type=jnp.float32)
    h2_ref[:, 0:NCLS] = h2
    h2_ref[:, NCLS:NCLS + 1] = jnp.ones((BN, 1), jnp.float32)
    h2_ref[:, NCLS + 1:W2] = jnp.zeros((BN, W2 - NCLS - 1), jnp.float32)
    fo_ref[0, 0, :] = jnp.dot(h2, ao_ref[0, 0:NCLS])
    fo_ref[0, 1, :] = jnp.dot(h2, ao_ref[0, NCLS:2 * NCLS])


def _dense2(agg1, W_out, a_out2d):
    return pl.pallas_call(
        _dense2_body,
        grid=(N // BN,),
        in_specs=[
            pl.BlockSpec((HEADS, BN, W1), lambda i: (0, i, 0)),
            pl.BlockSpec((HEADS * HID, NCLS), lambda i: (0, 0)),
            pl.BlockSpec((1, 2 * NCLS), lambda i: (0, 0)),
        ],
        out_specs=[
            pl.BlockSpec((BN, W2), lambda i: (i, 0)),
            pl.BlockSpec((1, 2, BN), lambda i: (i, 0, 0)),
        ],
        out_shape=[
            jax.ShapeDtypeStruct((N, W2), jnp.float32),
            jax.ShapeDtypeStruct((NBLK, 2, BN), jnp.float32),
        ],
    )(agg1, W_out, a_out2d)


def _edges2_body(h_hbm, fo_hbm, src_hbm, dst_hbm, out_hbm,
                 f1_vm, f2_vm, srcb, dstb, rows, wbuf, agg_sh):
    c = lax.axis_index("c")
    s = lax.axis_index("s")
    ebase = (c * NS + s) * ET2
    rbase = s * SPAN

    for k in range(NBLK):
        pltpu.sync_copy(fo_hbm.at[pl.ds(k * 2 * BN, BN)],
                        f1_vm.at[pl.ds(k * BN, BN)])
        pltpu.sync_copy(fo_hbm.at[pl.ds((k * 2 + 1) * BN, BN)],
                        f2_vm.at[pl.ds(k * BN, BN)])

    def zr(e, carry):
        for k in range(W2 // 16):
            rows[e, pl.ds(k * 16, 16)] = jnp.zeros((16,), jnp.float32)
        return carry
    lax.fori_loop(0, C2, zr, 0)

    @pl.when(s < NS - 1)
    def _():
        pltpu.sync_copy(rows, agg_sh.at[pl.ds(rbase, C2)])
        pltpu.sync_copy(rows.at[pl.ds(0, SPAN - C2)],
                        agg_sh.at[pl.ds(rbase + C2, SPAN - C2)])

    @pl.when(s == NS - 1)
    def _():
        pltpu.sync_copy(rows, agg_sh.at[pl.ds(15 * SPAN, TAIL)])
    plsc.subcore_barrier()

    def per_chunk(k, carry2):
        e0 = ebase + k * C2
        pltpu.sync_copy(src_hbm.at[pl.ds(e0, C2)], srcb)
        pltpu.sync_copy(dst_hbm.at[pl.ds(e0, C2)], dstb)

        def wloop(j, carry3):
            sv = srcb[pl.ds(j * 16, 16)]
            dv = dstb[pl.ds(j * 16, 16)]
            z = plsc.load_gather(f1_vm, [sv]) + plsc.load_gather(f2_vm, [dv])
            wbuf[pl.ds(j * 16, 16)] = jnp.exp(jnp.minimum(-z, -SLOPE * z))
            return carry3
        lax.fori_loop(0, C2 // 16, wloop, 0)

        pltpu.sync_copy(h_hbm.at[dstb], rows)

        def eloop(e, carry3):
            wv = plsc.load_gather(wbuf, [jnp.full((16,), e, jnp.int32)])
            for k2 in range(W2 // 16):
                rows[e, pl.ds(k2 * 16, 16)] = (
                    rows[e, pl.ds(k2 * 16, 16)] * wv)
            return carry3
        lax.fori_loop(0, C2, eloop, 0)

        pltpu.sync_copy(rows, agg_sh.at[srcb], add=True)
        return carry2
    lax.fori_loop(0, ET2 // C2, per_chunk, 0)
    plsc.subcore_barrier()

    @pl.when(s < NS - 1)
    def _():
        pltpu.sync_copy(agg_sh.at[pl.ds(rbase, SPAN)],
                        out_hbm.at[pl.ds(c * N + rbase, SPAN)])

    @pl.when(s == NS - 1)
    def _():
        pltpu.sync_copy(agg_sh.at[pl.ds(15 * SPAN, TAIL)],
                        out_hbm.at[pl.ds(c * N + 15 * SPAN, TAIL)])


def _edges2(h2aug, fo_flat, src, dst):
    return pl.kernel(
        _edges2_body,
        mesh=_MESH,
        compiler_params=pltpu.CompilerParams(needs_layout_passes=False, use_tc_tiling_on_sc=False),
        out_type=jax.ShapeDtypeStruct((NC * N, W2), jnp.float32),
        scratch_types=[
            pltpu.VMEM((N,), jnp.float32),
            pltpu.VMEM((N,), jnp.float32),
            pltpu.VMEM((C2,), jnp.int32),
            pltpu.VMEM((C2,), jnp.int32),
            pltpu.VMEM((C2, W2), jnp.float32),
            pltpu.VMEM((C2,), jnp.float32),
            pltpu.VMEM_SHARED((N, W2), jnp.float32),
        ],
    )(h2aug, fo_flat, src, dst)


def _final_body(p0_ref, p1_ref, o_ref):
    t = p0_ref[...] + p1_ref[...]
    agg = t[:, 0:NCLS]
    rs = t[:, NCLS:NCLS + 1]
    o = _elu(agg / (rs + 1e-16))
    o_ref[...] = jax.nn.log_softmax(o, axis=-1)


def _final(p):
    nblk = N // BN
    return pl.pallas_call(
        _final_body,
        grid=(nblk,),
        in_specs=[
            pl.BlockSpec((BN, W2), lambda i: (i, 0)),
            pl.BlockSpec((BN, W2), lambda i: (i + nblk, 0)),
        ],
        out_specs=pl.BlockSpec((BN, NCLS), lambda i: (i, 0)),
        out_shape=jax.ShapeDtypeStruct((N, NCLS), jnp.float32),
    )(p, p)


def kernel(x, adj, Wh, ah, W_out, a_out):
    src = adj[0]
    dst = adj[1]
    h1aug, f1, f2 = _dense1(x, Wh, ah)
    agg1 = _edges1(h1aug.reshape(HEADS * N, W1),
                   f1.reshape(NBLK * HEADS * BN),
                   f2.reshape(NBLK * HEADS * BN), src, dst)
    h2aug, fo = _dense2(agg1.reshape(HEADS, N, W1), W_out,
                        a_out.reshape(1, 2 * NCLS))
    p = _edges2(h2aug, fo.reshape(NBLK * 2 * BN), src, dst)
    return _final(p)


# speedup vs baseline: 1.3353x; 1.3353x over previous
import jax
import jax.numpy as jnp
from jax import lax
from jax.experimental import pallas as pl
from jax.experimental.pallas import tpu as pltpu
from jax.experimental.pallas import tpu_sc as plsc

N = 10000
E = 320000
F_IN = 128
HID = 64
HEADS = 8
NCLS = 40
SLOPE = 0.2

W1 = 80
W2 = 48
NC = 2
NS = 16
BN = 1000
SPAN = 640
TAIL = N - 15 * SPAN

NBLK = N // BN
C1 = 400
ET1 = E // NS
C2 = 400
ET2 = E // (NC * NS)

_MESH = plsc.VectorSubcoreMesh(core_axis_name="c", subcore_axis_name="s")
_SC_PARAMS = pltpu.CompilerParams(
    needs_layout_passes=False, use_tc_tiling_on_sc=False)


def _elu(v):
    return jnp.where(v > 0, v, jnp.exp(jnp.minimum(v, 0.0)) - 1.0)


def _dense1_body(x_ref, wh_ref, ah_ref, h_ref, f1_ref, f2_ref):
    xb = x_ref[...]
    for h in range(HEADS):
        hm = jnp.dot(xb, wh_ref[h], preferred_element_type=jnp.float32)
        h_ref[h, :, 0:HID] = hm
        h_ref[h, :, HID:HID + 1] = jnp.ones((BN, 1), jnp.float32)
        h_ref[h, :, HID + 1:W1] = jnp.zeros((BN, W1 - HID - 1), jnp.float32)
        f1_ref[0, h, :] = jnp.dot(hm, ah_ref[h, 0:HID])
        f2_ref[0, h, :] = jnp.dot(hm, ah_ref[h, HID:2 * HID])


def _dense1(x, Wh, ah):
    return pl.pallas_call(
        _dense1_body,
        grid=(N // BN,),
        in_specs=[
            pl.BlockSpec((BN, F_IN), lambda i: (i, 0)),
            pl.BlockSpec((HEADS, F_IN, HID), lambda i: (0, 0, 0)),
            pl.BlockSpec((HEADS, 2 * HID), lambda i: (0, 0)),
        ],
        out_specs=[
            pl.BlockSpec((HEADS, BN, W1), lambda i: (0, i, 0)),
            pl.BlockSpec((1, HEADS, BN), lambda i: (i, 0, 0)),
            pl.BlockSpec((1, HEADS, BN), lambda i: (i, 0, 0)),
        ],
        out_shape=[
            jax.ShapeDtypeStruct((HEADS, N, W1), jnp.float32),
            jax.ShapeDtypeStruct((NBLK, HEADS, BN), jnp.float32),
            jax.ShapeDtypeStruct((NBLK, HEADS, BN), jnp.float32),
        ],
    )(x, Wh, ah)


def _wpre_body(f1_hbm, f2_hbm, src_hbm, dst_hbm, w_hbm,
               f1_vm, f2_vm, srcA, dstA, wb0, wb1, semw0, semw1):
    c = lax.axis_index("c")
    s = lax.axis_index("s")
    NCH = ET1 // C1
    wb = (wb0, wb1)
    semw = (semw0, semw1)

    pltpu.sync_copy(src_hbm.at[pl.ds(s * NCH, NCH)], srcA)
    pltpu.sync_copy(dst_hbm.at[pl.ds(s * NCH, NCH)], dstA)

    def per_head(i, carry):
        h = c * (HEADS // NC) + i
        for k in range(NBLK):
            pltpu.sync_copy(f1_hbm.at[pl.ds((k * HEADS + h) * BN, BN)],
                            f1_vm.at[pl.ds(k * BN, BN)])
            pltpu.sync_copy(f2_hbm.at[pl.ds((k * HEADS + h) * BN, BN)],
                            f2_vm.at[pl.ds(k * BN, BN)])
        wbase = h * E + s * ET1

        def pair_loop(k, carry2):
            for b in range(2):
                kk = 2 * k + b

                @pl.when(k > 0)
                def _():
                    pltpu.make_async_copy(
                        wb[b], w_hbm.at[pl.ds(wbase, C1)], semw[b]).wait()

                def wloop(j, carry3):
                    sv = srcA[kk, pl.ds(j * 16, 16)]
                    dv = dstA[kk, pl.ds(j * 16, 16)]
                    z = (plsc.load_gather(f1_vm, [sv])
                         + plsc.load_gather(f2_vm, [dv]))
                    wb[b][pl.ds(j * 16, 16)] = jnp.exp(
                        jnp.minimum(-z, -SLOPE * z))
                    return carry3
                lax.fori_loop(0, C1 // 16, wloop, 0, unroll=2)
                pltpu.async_copy(
                    wb[b], w_hbm.at[pl.ds(wbase + kk * C1, C1)], semw[b])
            return carry2
        lax.fori_loop(0, NCH // 2, pair_loop, 0)
        for b in range(2):
            pltpu.make_async_copy(
                wb[b], w_hbm.at[pl.ds(wbase, C1)], semw[b]).wait()
        return carry
    lax.fori_loop(0, HEADS // NC, per_head, 0)


def _wpre(f1_flat, f2_flat, src2d, dst2d):
    return pl.kernel(
        _wpre_body,
        mesh=_MESH,
        compiler_params=_SC_PARAMS,
        out_type=jax.ShapeDtypeStruct((HEADS * E,), jnp.float32),
        scratch_types=[
            pltpu.VMEM((N,), jnp.float32),
            pltpu.VMEM((N,), jnp.float32),
            pltpu.VMEM((ET1 // C1, C1), jnp.int32),
            pltpu.VMEM((ET1 // C1, C1), jnp.int32),
            pltpu.VMEM((C1,), jnp.float32),
            pltpu.VMEM((C1,), jnp.float32),
            pltpu.SemaphoreType.DMA,
            pltpu.SemaphoreType.DMA,
        ],
    )(f1_flat, f2_flat, src2d, dst2d)


def _edges1_body(h_hbm, w_hbm, src_hbm, dst_hbm, out_hbm,
                 srcb0, srcb1, dstb0, dstb1, wq0, wq1, didx0, didx1,
                 rows0, rows1, seml0, seml1, semg0, semg1, sems0, sems1,
                 agg_sh):
    c = lax.axis_index("c")
    s = lax.axis_index("s")
    rbase = s * SPAN
    NCH = ET1 // C1
    srcb = (srcb0, srcb1)
    dstb = (dstb0, dstb1)
    wq = (wq0, wq1)
    didx = (didx0, didx1)
    rows = (rows0, rows1)
    seml = (seml0, seml1)
    semg = (semg0, semg1)
    sems = (sems0, sems1)
    ebase = s * NCH

    def issue_loads(kk, h, b):
        e0 = (ebase + kk) * C1
        pltpu.async_copy(src_hbm.at[pl.ds(e0, C1)], srcb[b], seml[b])
        pltpu.async_copy(dst_hbm.at[pl.ds(e0, C1)], dstb[b], seml[b])
        pltpu.async_copy(w_hbm.at[pl.ds(h * E + e0, C1)], wq[b], seml[b])

    def wait_loads(b):
        pltpu.make_async_copy(src_hbm.at[pl.ds(0, C1)], srcb[b],
                              seml[b]).wait()
        pltpu.make_async_copy(dst_hbm.at[pl.ds(0, C1)], dstb[b],
                              seml[b]).wait()
        pltpu.make_async_copy(w_hbm.at[pl.ds(0, C1)], wq[b], seml[b]).wait()

    def didx_gather(h, b):
        def dloop(j, carry3):
            dv = dstb[b][pl.ds(j * 16, 16)]
            didx[b][pl.ds(j * 16, 16)] = dv + h * N
            return carry3
        lax.fori_loop(0, C1 // 16, dloop, 0, unroll=2)
        pltpu.async_copy(h_hbm.at[didx[b]], rows[b], semg[b])

    def scale(b):
        def eloop(e2, carry3):
            for u in range(2):
                e = e2 * 2 + u
                wv = plsc.load_gather(wq[b], [jnp.full((16,), e, jnp.int32)])
                for k2 in range(W1 // 16):
                    rows[b][e, pl.ds(k2 * 16, 16)] = (
                        rows[b][e, pl.ds(k2 * 16, 16)] * wv)
            return carry3
        lax.fori_loop(0, C1 // 2, eloop, 0, unroll=2)

    def per_head(i, carry):
        h = c * (HEADS // NC) + i

        def zr(e, carry2):
            for k in range(W1 // 16):
                rows0[e, pl.ds(k * 16, 16)] = jnp.zeros((16,), jnp.float32)
            return carry2
        lax.fori_loop(0, C1, zr, 0)

        @pl.when(s < NS - 1)
        def _():
            pltpu.sync_copy(rows0, agg_sh.at[pl.ds(rbase, C1)])
            pltpu.sync_copy(rows0.at[pl.ds(0, SPAN - C1)],
                            agg_sh.at[pl.ds(rbase + C1, SPAN - C1)])

        @pl.when(s == NS - 1)
        def _():
            pltpu.sync_copy(rows0, agg_sh.at[pl.ds(15 * SPAN, TAIL)])
        plsc.subcore_barrier()

        for b in range(2):
            issue_loads(b, h, b)
        for b in range(2):
            wait_loads(b)
            didx_gather(h, b)

        def pair_loop(k, carry2):
            for b in range(2):
                pltpu.make_async_copy(h_hbm.at[didx[b]], rows[b],
                                      semg[b]).wait()
                scale(b)
                pltpu.async_copy(rows[b], agg_sh.at[srcb[b]], sems[b],
                                 add=True)

            @pl.when(k + 1 < NCH // 2)
            def _():
                for b in range(2):
                    kk2 = 2 * (k + 1) + b
                    pltpu.make_async_copy(rows[b], agg_sh.at[srcb[b]],
                                          sems[b]).wait()
                    issue_loads(kk2, h, b)
                for b in range(2):
                    wait_loads(b)
                    didx_gather(h, b)
            return carry2
        lax.fori_loop(0, NCH // 2, pair_loop, 0)
        for b in range(2):
            pltpu.make_async_copy(rows[b], agg_sh.at[srcb[b]],
                                  sems[b]).wait()
        plsc.subcore_barrier()

        @pl.when(s < NS - 1)
        def _():
            pltpu.sync_copy(agg_sh.at[pl.ds(rbase, SPAN)],
                            out_hbm.at[pl.ds(h * N + rbase, SPAN)])

        @pl.when(s == NS - 1)
        def _():
            pltpu.sync_copy(agg_sh.at[pl.ds(15 * SPAN, TAIL)],
                            out_hbm.at[pl.ds(h * N + 15 * SPAN, TAIL)])
        plsc.subcore_barrier()
        return carry
    lax.fori_loop(0, HEADS // NC, per_head, 0)


def _edges1(h_flat, w_flat, src_flat, dst_flat):
    return pl.kernel(
        _edges1_body,
        mesh=_MESH,
        compiler_params=_SC_PARAMS,
        out_type=jax.ShapeDtypeStruct((HEADS * N, W1), jnp.float32),
        scratch_types=[
            pltpu.VMEM((C1,), jnp.int32),
            pltpu.VMEM((C1,), jnp.int32),
            pltpu.VMEM((C1,), jnp.int32),
            pltpu.VMEM((C1,), jnp.int32),
            pltpu.VMEM((C1,), jnp.float32),
            pltpu.VMEM((C1,), jnp.float32),
            pltpu.VMEM((C1,), jnp.int32),
            pltpu.VMEM((C1,), jnp.int32),
            pltpu.VMEM((C1, W1), jnp.float32),
            pltpu.VMEM((C1, W1), jnp.float32),
            pltpu.SemaphoreType.DMA,
            pltpu.SemaphoreType.DMA,
            pltpu.SemaphoreType.DMA,
            pltpu.SemaphoreType.DMA,
            pltpu.SemaphoreType.DMA,
            pltpu.SemaphoreType.DMA,
            pltpu.VMEM_SHARED((N, W1), jnp.float32),
        ],
    )(h_flat, w_flat, src_flat, dst_flat)


def _dense2_body(agg_ref, wo_ref, ao_ref, h2_ref, fo_ref):
    parts = []
    for h in range(HEADS):
        a = agg_ref[h, :, 0:HID]
        rs = agg_ref[h, :, HID:HID + 1]
        parts.append(_elu(a / (rs + 1e-16)))
    h1 = jnp.concatenate(parts, axis=1)
    h2 = jnp.dot(h1, wo_ref[...], preferred_element_type=jnp.float32)
    h2_ref[:, 0:NCLS] = h2
    h2_ref[:, NCLS:NCLS + 1] = jnp.ones((BN, 1), jnp.float32)
    h2_ref[:, NCLS + 1:W2] = jnp.zeros((BN, W2 - NCLS - 1), jnp.float32)
    fo_ref[0, 0, :] = jnp.dot(h2, ao_ref[0, 0:NCLS])
    fo_ref[0, 1, :] = jnp.dot(h2, ao_ref[0, NCLS:2 * NCLS])


def _dense2(agg1, W_out, a_out2d):
    return pl.pallas_call(
        _dense2_body,
        grid=(N // BN,),
        in_specs=[
            pl.BlockSpec((HEADS, BN, W1), lambda i: (0, i, 0)),
            pl.BlockSpec((HEADS * HID, NCLS), lambda i: (0, 0)),
            pl.BlockSpec((1, 2 * NCLS), lambda i: (0, 0)),
        ],
        out_specs=[
            pl.BlockSpec((BN, W2), lambda i: (i, 0)),
            pl.BlockSpec((1, 2, BN), lambda i: (i, 0, 0)),
        ],
        out_shape=[
            jax.ShapeDtypeStruct((N, W2), jnp.float32),
            jax.ShapeDtypeStruct((NBLK, 2, BN), jnp.float32),
        ],
    )(agg1, W_out, a_out2d)


def _edges2_body(h_hbm, fo_hbm, src_hbm, dst_hbm, out_hbm,
                 f1_vm, f2_vm, srcA, dstA, rows0, rows1,
                 wbuf0, wbuf1, semg0, semg1, sems0, sems1, agg_sh):
    c = lax.axis_index("c")
    s = lax.axis_index("s")
    rbase = s * SPAN
    NCH = ET2 // C2
    rows = (rows0, rows1)
    wbuf = (wbuf0, wbuf1)
    semg = (semg0, semg1)
    sems = (sems0, sems1)

    wid = c * NS + s
    pltpu.sync_copy(src_hbm.at[pl.ds(wid * NCH, NCH)], srcA)
    pltpu.sync_copy(dst_hbm.at[pl.ds(wid * NCH, NCH)], dstA)
    for k in range(NBLK):
        pltpu.sync_copy(fo_hbm.at[pl.ds(k * 2 * BN, BN)],
                        f1_vm.at[pl.ds(k * BN, BN)])
        pltpu.sync_copy(fo_hbm.at[pl.ds((k * 2 + 1) * BN, BN)],
                        f2_vm.at[pl.ds(k * BN, BN)])

    def w_only(kk, b):
        def wloop(j, carry3):
            sv = srcA[kk, pl.ds(j * 16, 16)]
            dv = dstA[kk, pl.ds(j * 16, 16)]
            z = plsc.load_gather(f1_vm, [sv]) + plsc.load_gather(f2_vm, [dv])
            wbuf[b][pl.ds(j * 16, 16)] = jnp.exp(jnp.minimum(-z, -SLOPE * z))
            return carry3
        lax.fori_loop(0, C2 // 16, wloop, 0, unroll=2)

    def scale(b):
        def eloop(e2, carry3):
            for u in range(2):
                e = e2 * 2 + u
                wv = plsc.load_gather(wbuf[b], [jnp.full((16,), e, jnp.int32)])
                for k2 in range(W2 // 16):
                    rows[b][e, pl.ds(k2 * 16, 16)] = (
                        rows[b][e, pl.ds(k2 * 16, 16)] * wv)
            return carry3
        lax.fori_loop(0, C2 // 2, eloop, 0, unroll=2)

    def zr(e, carry):
        for k in range(W2 // 16):
            rows0[e, pl.ds(k * 16, 16)] = jnp.zeros((16,), jnp.float32)
        return carry
    lax.fori_loop(0, C2, zr, 0)

    @pl.when(s < NS - 1)
    def _():
        pltpu.sync_copy(rows0, agg_sh.at[pl.ds(rbase, C2)])
        pltpu.sync_copy(rows0.at[pl.ds(0, SPAN - C2)],
                        agg_sh.at[pl.ds(rbase + C2, SPAN - C2)])

    @pl.when(s == NS - 1)
    def _():
        pltpu.sync_copy(rows0, agg_sh.at[pl.ds(15 * SPAN, TAIL)])
    plsc.subcore_barrier()

    for b in range(2):
        w_only(b, b)
        pltpu.async_copy(h_hbm.at[dstA.at[b]], rows[b], semg[b])

    def pair_loop(k, carry2):
        for b in range(2):
            kk = 2 * k + b
            pltpu.make_async_copy(h_hbm.at[dstA.at[0]], rows[b],
                                  semg[b]).wait()
            scale(b)
            pltpu.async_copy(rows[b], agg_sh.at[srcA.at[kk]], sems[b],
                             add=True)

        @pl.when(k + 1 < NCH // 2)
        def _():
            for b in range(2):
                kk2 = 2 * (k + 1) + b
                w_only(kk2, b)
                pltpu.make_async_copy(rows[b], agg_sh.at[srcA.at[0]],
                                      sems[b]).wait()
                pltpu.async_copy(h_hbm.at[dstA.at[kk2]], rows[b], semg[b])
        return carry2
    lax.fori_loop(0, NCH // 2, pair_loop, 0)
    for b in range(2):
        pltpu.make_async_copy(rows[b], agg_sh.at[srcA.at[0]], sems[b]).wait()
    plsc.subcore_barrier()

    @pl.when(s < NS - 1)
    def _():
        pltpu.sync_copy(agg_sh.at[pl.ds(rbase, SPAN)],
                        out_hbm.at[pl.ds(c * N + rbase, SPAN)])

    @pl.when(s == NS - 1)
    def _():
        pltpu.sync_copy(agg_sh.at[pl.ds(15 * SPAN, TAIL)],
                        out_hbm.at[pl.ds(c * N + 15 * SPAN, TAIL)])


def _edges2(h2aug, fo_flat, src2d, dst2d):
    return pl.kernel(
        _edges2_body,
        mesh=_MESH,
        compiler_params=_SC_PARAMS,
        out_type=jax.ShapeDtypeStruct((NC * N, W2), jnp.float32),
        scratch_types=[
            pltpu.VMEM((N,), jnp.float32),
            pltpu.VMEM((N,), jnp.float32),
            pltpu.VMEM((ET2 // C2, C2), jnp.int32),
            pltpu.VMEM((ET2 // C2, C2), jnp.int32),
            pltpu.VMEM((C2, W2), jnp.float32),
            pltpu.VMEM((C2, W2), jnp.float32),
            pltpu.VMEM((C2,), jnp.float32),
            pltpu.VMEM((C2,), jnp.float32),
            pltpu.SemaphoreType.DMA,
            pltpu.SemaphoreType.DMA,
            pltpu.SemaphoreType.DMA,
            pltpu.SemaphoreType.DMA,
            pltpu.VMEM_SHARED((N, W2), jnp.float32),
        ],
    )(h2aug, fo_flat, src2d, dst2d)


def _final_body(p0_ref, p1_ref, o_ref):
    t = p0_ref[...] + p1_ref[...]
    agg = t[:, 0:NCLS]
    rs = t[:, NCLS:NCLS + 1]
    o = _elu(agg / (rs + 1e-16))
    o_ref[...] = jax.nn.log_softmax(o, axis=-1)


def _final(p):
    nblk = N // BN
    return pl.pallas_call(
        _final_body,
        grid=(nblk,),
        in_specs=[
            pl.BlockSpec((BN, W2), lambda i: (i, 0)),
            pl.BlockSpec((BN, W2), lambda i: (i + nblk, 0)),
        ],
        out_specs=pl.BlockSpec((BN, NCLS), lambda i: (i, 0)),
        out_shape=jax.ShapeDtypeStruct((N, NCLS), jnp.float32),
    )(p, p)


def kernel(x, adj, Wh, ah, W_out, a_out):
    src = adj[0]
    dst = adj[1]
    src2d = src.reshape(E // C1, C1)
    dst2d = dst.reshape(E // C1, C1)
    h1aug, f1, f2 = _dense1(x, Wh, ah)
    wall = _wpre(f1.reshape(NBLK * HEADS * BN),
                 f2.reshape(NBLK * HEADS * BN), src2d, dst2d)
    agg1 = _edges1(h1aug.reshape(HEADS * N, W1), wall, src, dst)
    h2aug, fo = _dense2(agg1.reshape(HEADS, N, W1), W_out,
                        a_out.reshape(1, 2 * NCLS))
    p = _edges2(h2aug, fo.reshape(NBLK * 2 * BN), src2d, dst2d)
    return _final(p)


# speedup vs baseline: 1.4972x; 1.1213x over previous
import jax
import jax.numpy as jnp
from jax import lax
from jax.experimental import pallas as pl
from jax.experimental.pallas import tpu as pltpu
from jax.experimental.pallas import tpu_sc as plsc

N = 10000
E = 320000
F_IN = 128
HID = 64
HEADS = 8
NCLS = 40
SLOPE = 0.2

W1 = 96
W2 = 48
NC = 2
NS = 16
BN = 1000
SPAN = 640
TAIL = N - 15 * SPAN

NBLK = N // BN
C1 = 400
ET1 = E // NS
C2 = 400
ET2 = E // (NC * NS)

_MESH = plsc.VectorSubcoreMesh(core_axis_name="c", subcore_axis_name="s")
_SC_PARAMS = pltpu.CompilerParams(
    needs_layout_passes=False, use_tc_tiling_on_sc=False)


def _elu(v):
    return jnp.where(v > 0, v, jnp.exp(jnp.minimum(v, 0.0)) - 1.0)


def _densef_body(x_ref, wh_ref, ah_ref, f_ref):
    xb = x_ref[...]
    vs = []
    for h in range(HEADS):
        vs.append(jnp.dot(wh_ref[h], ah_ref[h, 0:HID]))
    for h in range(HEADS):
        vs.append(jnp.dot(wh_ref[h], ah_ref[h, HID:2 * HID]))
    v = jnp.stack(vs, axis=0)
    f_ref[0, :, :] = lax.dot_general(
        v, xb, (((1,), (1,)), ((), ())),
        preferred_element_type=jnp.float32)


def _densef(x, Wh, ah):
    return pl.pallas_call(
        _densef_body,
        grid=(N // BN,),
        in_specs=[
            pl.BlockSpec((BN, F_IN), lambda i: (i, 0)),
            pl.BlockSpec((HEADS, F_IN, HID), lambda i: (0, 0, 0)),
            pl.BlockSpec((HEADS, 2 * HID), lambda i: (0, 0)),
        ],
        out_specs=pl.BlockSpec((1, 2 * HEADS, BN), lambda i: (i, 0, 0)),
        out_shape=jax.ShapeDtypeStruct((NBLK, 2 * HEADS, BN), jnp.float32),
    )(x, Wh, ah)


BN1 = 2000


def _dense1_body(x_ref, wh_ref, h_ref):
    xb = x_ref[...]
    for h in range(HEADS):
        hm = jnp.dot(xb, wh_ref[h], preferred_element_type=jnp.float32)
        h_ref[h, :, 0:HID] = hm.astype(jnp.bfloat16)
        h_ref[h, :, HID:HID + 1] = jnp.ones((BN1, 1), jnp.bfloat16)
        h_ref[h, :, HID + 1:W1] = jnp.zeros((BN1, W1 - HID - 1), jnp.bfloat16)


def _dense1(x, Wh):
    return pl.pallas_call(
        _dense1_body,
        grid=(N // BN1,),
        in_specs=[
            pl.BlockSpec((BN1, F_IN), lambda i: (i, 0)),
            pl.BlockSpec((HEADS, F_IN, HID), lambda i: (0, 0, 0)),
        ],
        out_specs=pl.BlockSpec((HEADS, BN1, W1), lambda i: (0, i, 0)),
        out_shape=jax.ShapeDtypeStruct((HEADS, N, W1), jnp.bfloat16),
    )(x, Wh)


def _wpre_body(ff_hbm, src_hbm, dst_hbm, w_hbm,
               f1_vm, f2_vm, srcA, dstA, wb0, wb1, semw0, semw1):
    c = lax.axis_index("c")
    s = lax.axis_index("s")
    NCH = ET1 // C1
    wb = (wb0, wb1)
    semw = (semw0, semw1)

    pltpu.sync_copy(src_hbm.at[pl.ds(s * NCH, NCH)], srcA)
    pltpu.sync_copy(dst_hbm.at[pl.ds(s * NCH, NCH)], dstA)

    def per_head(i, carry):
        h = c * (HEADS // NC) + i
        for k in range(NBLK):
            pltpu.sync_copy(
                ff_hbm.at[pl.ds((k * 2 * HEADS + h) * BN, BN)],
                f1_vm.at[pl.ds(k * BN, BN)])
            pltpu.sync_copy(
                ff_hbm.at[pl.ds((k * 2 * HEADS + HEADS + h) * BN, BN)],
                f2_vm.at[pl.ds(k * BN, BN)])
        wbase = h * E + s * ET1

        def pair_loop(k, carry2):
            for b in range(2):
                kk = 2 * k + b

                @pl.when(k > 0)
                def _():
                    pltpu.make_async_copy(
                        wb[b], w_hbm.at[pl.ds(wbase, C1)], semw[b]).wait()

                def wloop(j, carry3):
                    sv = srcA[kk, pl.ds(j * 16, 16)]
                    dv = dstA[kk, pl.ds(j * 16, 16)]
                    z = (plsc.load_gather(f1_vm, [sv])
                         + plsc.load_gather(f2_vm, [dv]))
                    wb[b][pl.ds(j * 16, 16)] = jnp.exp(
                        jnp.minimum(-z, -SLOPE * z))
                    return carry3
                lax.fori_loop(0, C1 // 16, wloop, 0, unroll=2)
                pltpu.async_copy(
                    wb[b], w_hbm.at[pl.ds(wbase + kk * C1, C1)], semw[b])
            return carry2
        lax.fori_loop(0, NCH // 2, pair_loop, 0)
        for b in range(2):
            pltpu.make_async_copy(
                wb[b], w_hbm.at[pl.ds(wbase, C1)], semw[b]).wait()
        return carry
    lax.fori_loop(0, HEADS // NC, per_head, 0)


def _wpre(ff_flat, src2d, dst2d):
    return pl.kernel(
        _wpre_body,
        mesh=_MESH,
        compiler_params=_SC_PARAMS,
        out_type=jax.ShapeDtypeStruct((HEADS * E,), jnp.float32),
        scratch_types=[
            pltpu.VMEM((N,), jnp.float32),
            pltpu.VMEM((N,), jnp.float32),
            pltpu.VMEM((ET1 // C1, C1), jnp.int32),
            pltpu.VMEM((ET1 // C1, C1), jnp.int32),
            pltpu.VMEM((C1,), jnp.float32),
            pltpu.VMEM((C1,), jnp.float32),
            pltpu.SemaphoreType.DMA,
            pltpu.SemaphoreType.DMA,
        ],
    )(ff_flat, src2d, dst2d)


def _edges1_body(h_hbm, w_hbm, src_hbm, dst_hbm, out_hbm,
                 srcb0, srcb1, dstb0, dstb1, wq0, wq1, didx0, didx1,
                 rows0, rows1, seml0, seml1, semg0, semg1, sems0, sems1,
                 agg_sh):
    c = lax.axis_index("c")
    s = lax.axis_index("s")
    rbase = s * SPAN
    NCH = ET1 // C1
    srcb = (srcb0, srcb1)
    dstb = (dstb0, dstb1)
    wq = (wq0, wq1)
    didx = (didx0, didx1)
    rows = (rows0, rows1)
    seml = (seml0, seml1)
    semg = (semg0, semg1)
    sems = (sems0, sems1)
    ebase = s * NCH

    def issue_loads(kk, h, b):
        e0 = (ebase + kk) * C1
        pltpu.async_copy(src_hbm.at[pl.ds(e0, C1)], srcb[b], seml[b])
        pltpu.async_copy(dst_hbm.at[pl.ds(e0, C1)], dstb[b], seml[b])
        pltpu.async_copy(w_hbm.at[pl.ds(h * E + e0, C1)], wq[b], seml[b])

    def wait_loads(b):
        pltpu.make_async_copy(src_hbm.at[pl.ds(0, C1)], srcb[b],
                              seml[b]).wait()
        pltpu.make_async_copy(dst_hbm.at[pl.ds(0, C1)], dstb[b],
                              seml[b]).wait()
        pltpu.make_async_copy(w_hbm.at[pl.ds(0, C1)], wq[b], seml[b]).wait()

    def didx_gather(h, b):
        def dloop(j, carry3):
            dv = dstb[b][pl.ds(j * 16, 16)]
            didx[b][pl.ds(j * 16, 16)] = dv + h * N
            return carry3
        lax.fori_loop(0, C1 // 16, dloop, 0, unroll=2)
        pltpu.async_copy(h_hbm.at[didx[b]], rows[b], semg[b])

    def scale(b):
        def eloop(e2, carry3):
            for u in range(2):
                e = e2 * 2 + u
                wv = plsc.load_gather(wq[b], [jnp.full((16,), e, jnp.int32)])
                wb16 = plsc.pack(wv, wv, format=plsc.PackFormat.INTERLEAVED)
                for k2 in range(W1 // 32):
                    rows[b][e, pl.ds(k2 * 32, 32)] = (
                        rows[b][e, pl.ds(k2 * 32, 32)] * wb16)
            return carry3
        lax.fori_loop(0, C1 // 2, eloop, 0, unroll=2)

    def per_head(i, carry):
        h = c * (HEADS // NC) + i

        def zr(e, carry2):
            for k in range(W1 // 32):
                rows0[e, pl.ds(k * 32, 32)] = jnp.zeros((32,), jnp.bfloat16)
            return carry2
        lax.fori_loop(0, C1, zr, 0)

        @pl.when(s < NS - 1)
        def _():
            pltpu.sync_copy(rows0, agg_sh.at[pl.ds(rbase, C1)])
            pltpu.sync_copy(rows0.at[pl.ds(0, SPAN - C1)],
                            agg_sh.at[pl.ds(rbase + C1, SPAN - C1)])

        @pl.when(s == NS - 1)
        def _():
            pltpu.sync_copy(rows0, agg_sh.at[pl.ds(15 * SPAN, TAIL)])
        plsc.subcore_barrier()

        for b in range(2):
            issue_loads(b, h, b)
        for b in range(2):
            wait_loads(b)
            didx_gather(h, b)

        def pair_loop(k, carry2):
            for b in range(2):
                pltpu.make_async_copy(h_hbm.at[didx[b]], rows[b],
                                      semg[b]).wait()
                scale(b)
                pltpu.async_copy(rows[b], agg_sh.at[srcb[b]], sems[b],
                                 add=True)

            @pl.when(k + 1 < NCH // 2)
            def _():
                for b in range(2):
                    kk2 = 2 * (k + 1) + b
                    pltpu.make_async_copy(rows[b], agg_sh.at[srcb[b]],
                                          sems[b]).wait()
                    issue_loads(kk2, h, b)
                for b in range(2):
                    wait_loads(b)
                    didx_gather(h, b)
            return carry2
        lax.fori_loop(0, NCH // 2, pair_loop, 0)
        for b in range(2):
            pltpu.make_async_copy(rows[b], agg_sh.at[srcb[b]],
                                  sems[b]).wait()
        plsc.subcore_barrier()

        @pl.when(s < NS - 1)
        def _():
            pltpu.sync_copy(agg_sh.at[pl.ds(rbase, SPAN)],
                            out_hbm.at[pl.ds(h * N + rbase, SPAN)])

        @pl.when(s == NS - 1)
        def _():
            pltpu.sync_copy(agg_sh.at[pl.ds(15 * SPAN, TAIL)],
                            out_hbm.at[pl.ds(h * N + 15 * SPAN, TAIL)])
        plsc.subcore_barrier()
        return carry
    lax.fori_loop(0, HEADS // NC, per_head, 0)


def _edges1(h_flat, w_flat, src_flat, dst_flat):
    return pl.kernel(
        _edges1_body,
        mesh=_MESH,
        compiler_params=_SC_PARAMS,
        out_type=jax.ShapeDtypeStruct((HEADS * N, W1), jnp.bfloat16),
        scratch_types=[
            pltpu.VMEM((C1,), jnp.int32),
            pltpu.VMEM((C1,), jnp.int32),
            pltpu.VMEM((C1,), jnp.int32),
            pltpu.VMEM((C1,), jnp.int32),
            pltpu.VMEM((C1,), jnp.float32),
            pltpu.VMEM((C1,), jnp.float32),
            pltpu.VMEM((C1,), jnp.int32),
            pltpu.VMEM((C1,), jnp.int32),
            pltpu.VMEM((C1, W1), jnp.bfloat16),
            pltpu.VMEM((C1, W1), jnp.bfloat16),
            pltpu.SemaphoreType.DMA,
            pltpu.SemaphoreType.DMA,
            pltpu.SemaphoreType.DMA,
            pltpu.SemaphoreType.DMA,
            pltpu.SemaphoreType.DMA,
            pltpu.SemaphoreType.DMA,
            pltpu.VMEM_SHARED((N, W1), jnp.bfloat16),
        ],
    )(h_flat, w_flat, src_flat, dst_flat)


def _dense2_body(agg_ref, wo_ref, ao_ref, h2_ref, fo_ref):
    parts = []
    for h in range(HEADS):
        a = agg_ref[h, :, 0:HID].astype(jnp.float32)
        rs = agg_ref[h, :, HID:HID + 1].astype(jnp.float32)
        parts.append(_elu(a / (rs + 1e-16)))
    h1 = jnp.concatenate(parts, axis=1)
    h2 = jnp.dot(h1, wo_ref[...], preferred_element_type=jnp.float32)
    h2_ref[:, 0:NCLS] = h2
    h2_ref[:, NCLS:NCLS + 1] = jnp.ones((BN, 1), jnp.float32)
    h2_ref[:, NCLS + 1:W2] = jnp.zeros((BN, W2 - NCLS - 1), jnp.float32)
    fo_ref[0, 0, :] = jnp.dot(h2, ao_ref[0, 0:NCLS])
    fo_ref[0, 1, :] = jnp.dot(h2, ao_ref[0, NCLS:2 * NCLS])


def _dense2(agg1, W_out, a_out2d):
    return pl.pallas_call(
        _dense2_body,
        grid=(N // BN,),
        in_specs=[
            pl.BlockSpec((HEADS, BN, W1), lambda i: (0, i, 0)),
            pl.BlockSpec((HEADS * HID, NCLS), lambda i: (0, 0)),
            pl.BlockSpec((1, 2 * NCLS), lambda i: (0, 0)),
        ],
        out_specs=[
            pl.BlockSpec((BN, W2), lambda i: (i, 0)),
            pl.BlockSpec((1, 2, BN), lambda i: (i, 0, 0)),
        ],
        out_shape=[
            jax.ShapeDtypeStruct((N, W2), jnp.float32),
            jax.ShapeDtypeStruct((NBLK, 2, BN), jnp.float32),
        ],
    )(agg1, W_out, a_out2d)


def _edges2_body(h_hbm, fo_hbm, src_hbm, dst_hbm, out_hbm,
                 f1_vm, f2_vm, srcA, dstA, rows0, rows1,
                 wbuf0, wbuf1, semg0, semg1, sems0, sems1, agg_sh):
    c = lax.axis_index("c")
    s = lax.axis_index("s")
    rbase = s * SPAN
    NCH = ET2 // C2
    rows = (rows0, rows1)
    wbuf = (wbuf0, wbuf1)
    semg = (semg0, semg1)
    sems = (sems0, sems1)

    wid = c * NS + s
    pltpu.sync_copy(src_hbm.at[pl.ds(wid * NCH, NCH)], srcA)
    pltpu.sync_copy(dst_hbm.at[pl.ds(wid * NCH, NCH)], dstA)
    for k in range(NBLK):
        pltpu.sync_copy(fo_hbm.at[pl.ds(k * 2 * BN, BN)],
                        f1_vm.at[pl.ds(k * BN, BN)])
        pltpu.sync_copy(fo_hbm.at[pl.ds((k * 2 + 1) * BN, BN)],
                        f2_vm.at[pl.ds(k * BN, BN)])

    def w_only(kk, b):
        def wloop(j, carry3):
            sv = srcA[kk, pl.ds(j * 16, 16)]
            dv = dstA[kk, pl.ds(j * 16, 16)]
            z = plsc.load_gather(f1_vm, [sv]) + plsc.load_gather(f2_vm, [dv])
            wbuf[b][pl.ds(j * 16, 16)] = jnp.exp(jnp.minimum(-z, -SLOPE * z))
            return carry3
        lax.fori_loop(0, C2 // 16, wloop, 0, unroll=2)

    def scale(b):
        def eloop(e2, carry3):
            for u in range(2):
                e = e2 * 2 + u
                wv = plsc.load_gather(wbuf[b], [jnp.full((16,), e, jnp.int32)])
                for k2 in range(W2 // 16):
                    rows[b][e, pl.ds(k2 * 16, 16)] = (
                        rows[b][e, pl.ds(k2 * 16, 16)] * wv)
            return carry3
        lax.fori_loop(0, C2 // 2, eloop, 0, unroll=2)

    def zr(e, carry):
        for k in range(W2 // 16):
            rows0[e, pl.ds(k * 16, 16)] = jnp.zeros((16,), jnp.float32)
        return carry
    lax.fori_loop(0, C2, zr, 0)

    @pl.when(s < NS - 1)
    def _():
        pltpu.sync_copy(rows0, agg_sh.at[pl.ds(rbase, C2)])
        pltpu.sync_copy(rows0.at[pl.ds(0, SPAN - C2)],
                        agg_sh.at[pl.ds(rbase + C2, SPAN - C2)])

    @pl.when(s == NS - 1)
    def _():
        pltpu.sync_copy(rows0, agg_sh.at[pl.ds(15 * SPAN, TAIL)])
    plsc.subcore_barrier()

    for b in range(2):
        w_only(b, b)
        pltpu.async_copy(h_hbm.at[dstA.at[b]], rows[b], semg[b])

    def pair_loop(k, carry2):
        for b in range(2):
            kk = 2 * k + b
            pltpu.make_async_copy(h_hbm.at[dstA.at[0]], rows[b],
                                  semg[b]).wait()
            scale(b)
            pltpu.async_copy(rows[b], agg_sh.at[srcA.at[kk]], sems[b],
                             add=True)

        @pl.when(k + 1 < NCH // 2)
        def _():
            for b in range(2):
                kk2 = 2 * (k + 1) + b
                w_only(kk2, b)
                pltpu.make_async_copy(rows[b], agg_sh.at[srcA.at[0]],
                                      sems[b]).wait()
                pltpu.async_copy(h_hbm.at[dstA.at[kk2]], rows[b], semg[b])
        return carry2
    lax.fori_loop(0, NCH // 2, pair_loop, 0)
    for b in range(2):
        pltpu.make_async_copy(rows[b], agg_sh.at[srcA.at[0]], sems[b]).wait()
    plsc.subcore_barrier()

    @pl.when(s < NS - 1)
    def _():
        pltpu.sync_copy(agg_sh.at[pl.ds(rbase, SPAN)],
                        out_hbm.at[pl.ds(c * N + rbase, SPAN)])

    @pl.when(s == NS - 1)
    def _():
        pltpu.sync_copy(agg_sh.at[pl.ds(15 * SPAN, TAIL)],
                        out_hbm.at[pl.ds(c * N + 15 * SPAN, TAIL)])


def _edges2(h2aug, fo_flat, src2d, dst2d):
    return pl.kernel(
        _edges2_body,
        mesh=_MESH,
        compiler_params=_SC_PARAMS,
        out_type=jax.ShapeDtypeStruct((NC * N, W2), jnp.float32),
        scratch_types=[
            pltpu.VMEM((N,), jnp.float32),
            pltpu.VMEM((N,), jnp.float32),
            pltpu.VMEM((ET2 // C2, C2), jnp.int32),
            pltpu.VMEM((ET2 // C2, C2), jnp.int32),
            pltpu.VMEM((C2, W2), jnp.float32),
            pltpu.VMEM((C2, W2), jnp.float32),
            pltpu.VMEM((C2,), jnp.float32),
            pltpu.VMEM((C2,), jnp.float32),
            pltpu.SemaphoreType.DMA,
            pltpu.SemaphoreType.DMA,
            pltpu.SemaphoreType.DMA,
            pltpu.SemaphoreType.DMA,
            pltpu.VMEM_SHARED((N, W2), jnp.float32),
        ],
    )(h2aug, fo_flat, src2d, dst2d)


def _final_body(p0_ref, p1_ref, o_ref):
    t = p0_ref[...] + p1_ref[...]
    agg = t[:, 0:NCLS]
    rs = t[:, NCLS:NCLS + 1]
    o = _elu(agg / (rs + 1e-16))
    o_ref[...] = jax.nn.log_softmax(o, axis=-1)


def _final(p):
    nblk = N // BN
    return pl.pallas_call(
        _final_body,
        grid=(nblk,),
        in_specs=[
            pl.BlockSpec((BN, W2), lambda i: (i, 0)),
            pl.BlockSpec((BN, W2), lambda i: (i + nblk, 0)),
        ],
        out_specs=pl.BlockSpec((BN, NCLS), lambda i: (i, 0)),
        out_shape=jax.ShapeDtypeStruct((N, NCLS), jnp.float32),
    )(p, p)


def kernel(x, adj, Wh, ah, W_out, a_out):
    src = adj[0]
    dst = adj[1]
    src2d = src.reshape(E // C1, C1)
    dst2d = dst.reshape(E // C1, C1)
    ff = _densef(x, Wh, ah)
    wall = _wpre(ff.reshape(NBLK * 2 * HEADS * BN), src2d, dst2d)
    h1aug = _dense1(x, Wh)
    agg1 = _edges1(h1aug.reshape(HEADS * N, W1), wall, src, dst)
    h2aug, fo = _dense2(agg1.reshape(HEADS, N, W1), W_out,
                        a_out.reshape(1, 2 * NCLS))
    p = _edges2(h2aug, fo.reshape(NBLK * 2 * BN), src2d, dst2d)
    return _final(p)


# speedup vs baseline: 1.5933x; 1.0641x over previous
import jax
import jax.numpy as jnp
from jax import lax
from jax.experimental import pallas as pl
from jax.experimental.pallas import tpu as pltpu
from jax.experimental.pallas import tpu_sc as plsc

N = 10000
E = 320000
F_IN = 128
HID = 64
HEADS = 8
NCLS = 40
SLOPE = 0.2

W1 = 96
W2 = 48
NC = 2
NS = 16
BN = 1000
SPAN = 640
TAIL = N - 15 * SPAN

NBLK = N // BN
C1 = 400
ET1 = E // NS
C2 = 400
ET2 = E // (NC * NS)

_MESH = plsc.VectorSubcoreMesh(core_axis_name="c", subcore_axis_name="s")
_SC_PARAMS = pltpu.CompilerParams(
    needs_layout_passes=False, use_tc_tiling_on_sc=False)


def _elu(v):
    return jnp.where(v > 0, v, jnp.exp(jnp.minimum(v, 0.0)) - 1.0)


def _densef_body(x_ref, wh_ref, ah_ref, f_ref):
    xb = x_ref[...]
    vs = []
    for h in range(HEADS):
        vs.append(jnp.dot(wh_ref[h], ah_ref[h, 0:HID]))
    for h in range(HEADS):
        vs.append(jnp.dot(wh_ref[h], ah_ref[h, HID:2 * HID]))
    v = jnp.stack(vs, axis=0)
    f_ref[0, :, :] = lax.dot_general(
        v, xb, (((1,), (1,)), ((), ())),
        preferred_element_type=jnp.float32)


def _densef(x, Wh, ah):
    return pl.pallas_call(
        _densef_body,
        grid=(N // BN,),
        in_specs=[
            pl.BlockSpec((BN, F_IN), lambda i: (i, 0)),
            pl.BlockSpec((HEADS, F_IN, HID), lambda i: (0, 0, 0)),
            pl.BlockSpec((HEADS, 2 * HID), lambda i: (0, 0)),
        ],
        out_specs=pl.BlockSpec((1, 2 * HEADS, BN), lambda i: (i, 0, 0)),
        out_shape=jax.ShapeDtypeStruct((NBLK, 2 * HEADS, BN), jnp.float32),
    )(x, Wh, ah)


BN1 = 2000


def _dense1_body(x_ref, wh_ref, h_ref):
    xb = x_ref[...]
    for h in range(HEADS):
        hm = jnp.dot(xb, wh_ref[h], preferred_element_type=jnp.float32)
        h_ref[h, :, 0:HID] = hm.astype(jnp.bfloat16)
        h_ref[h, :, HID:HID + 1] = jnp.ones((BN1, 1), jnp.bfloat16)
        h_ref[h, :, HID + 1:W1] = jnp.zeros((BN1, W1 - HID - 1), jnp.bfloat16)


def _dense1(x, Wh):
    return pl.pallas_call(
        _dense1_body,
        grid=(N // BN1,),
        in_specs=[
            pl.BlockSpec((BN1, F_IN), lambda i: (i, 0)),
            pl.BlockSpec((HEADS, F_IN, HID), lambda i: (0, 0, 0)),
        ],
        out_specs=pl.BlockSpec((HEADS, BN1, W1), lambda i: (0, i, 0)),
        out_shape=jax.ShapeDtypeStruct((HEADS, N, W1), jnp.bfloat16),
    )(x, Wh)


def _wpre_body(ff_hbm, src_hbm, dst_hbm, w_hbm,
               f1_vm, f2_vm, srcA, dstA, wb0, wb1, semw0, semw1):
    c = lax.axis_index("c")
    s = lax.axis_index("s")
    NCH = ET1 // C1
    wb = (wb0, wb1)
    semw = (semw0, semw1)

    pltpu.sync_copy(src_hbm.at[pl.ds(s * NCH, NCH)], srcA)
    pltpu.sync_copy(dst_hbm.at[pl.ds(s * NCH, NCH)], dstA)

    def per_head(i, carry):
        h = c * (HEADS // NC) + i
        for k in range(NBLK):
            pltpu.sync_copy(
                ff_hbm.at[pl.ds((k * 2 * HEADS + h) * BN, BN)],
                f1_vm.at[pl.ds(k * BN, BN)])
            pltpu.sync_copy(
                ff_hbm.at[pl.ds((k * 2 * HEADS + HEADS + h) * BN, BN)],
                f2_vm.at[pl.ds(k * BN, BN)])
        wbase = h * E + s * ET1

        def pair_loop(k, carry2):
            for b in range(2):
                kk = 2 * k + b

                @pl.when(k > 0)
                def _():
                    pltpu.make_async_copy(
                        wb[b], w_hbm.at[pl.ds(wbase, C1)], semw[b]).wait()

                def wloop(j, carry3):
                    sv = srcA[kk, pl.ds(j * 16, 16)]
                    dv = dstA[kk, pl.ds(j * 16, 16)]
                    z = (plsc.load_gather(f1_vm, [sv])
                         + plsc.load_gather(f2_vm, [dv]))
                    wb[b][pl.ds(j * 16, 16)] = jnp.exp(
                        jnp.minimum(-z, -SLOPE * z))
                    return carry3
                lax.fori_loop(0, C1 // 16, wloop, 0, unroll=2)
                pltpu.async_copy(
                    wb[b], w_hbm.at[pl.ds(wbase + kk * C1, C1)], semw[b])
            return carry2
        lax.fori_loop(0, NCH // 2, pair_loop, 0)
        for b in range(2):
            pltpu.make_async_copy(
                wb[b], w_hbm.at[pl.ds(wbase, C1)], semw[b]).wait()
        return carry
    lax.fori_loop(0, HEADS // NC, per_head, 0)


def _wpre(ff_flat, src2d, dst2d):
    return pl.kernel(
        _wpre_body,
        mesh=_MESH,
        compiler_params=_SC_PARAMS,
        out_type=jax.ShapeDtypeStruct((HEADS * E,), jnp.float32),
        scratch_types=[
            pltpu.VMEM((N,), jnp.float32),
            pltpu.VMEM((N,), jnp.float32),
            pltpu.VMEM((ET1 // C1, C1), jnp.int32),
            pltpu.VMEM((ET1 // C1, C1), jnp.int32),
            pltpu.VMEM((C1,), jnp.float32),
            pltpu.VMEM((C1,), jnp.float32),
            pltpu.SemaphoreType.DMA,
            pltpu.SemaphoreType.DMA,
        ],
    )(ff_flat, src2d, dst2d)


def _edges1_body(h_hbm, w_hbm, src_hbm, dst_hbm, out_hbm,
                 srcA, dstA, wq0, wq1, didx0, didx1,
                 rows0, rows1, seml0, seml1, semg0, semg1, sems0, sems1,
                 agg_sh):
    c = lax.axis_index("c")
    s = lax.axis_index("s")
    rbase = s * SPAN
    NCH = ET1 // C1
    wq = (wq0, wq1)
    didx = (didx0, didx1)
    rows = (rows0, rows1)
    seml = (seml0, seml1)
    semg = (semg0, semg1)
    sems = (sems0, sems1)
    ebase = s * NCH

    pltpu.sync_copy(src_hbm.at[pl.ds(ebase, NCH)], srcA)
    pltpu.sync_copy(dst_hbm.at[pl.ds(ebase, NCH)], dstA)

    def issue_wq(kk, h, b):
        pltpu.async_copy(
            w_hbm.at[pl.ds(h * E + (ebase + kk) * C1, C1)], wq[b], seml[b])

    def wait_wq(b):
        pltpu.make_async_copy(w_hbm.at[pl.ds(0, C1)], wq[b], seml[b]).wait()

    def didx_gather(kk, h, b):
        def dloop(j, carry3):
            dv = dstA[kk, pl.ds(j * 16, 16)]
            didx[b][pl.ds(j * 16, 16)] = dv + h * N
            return carry3
        lax.fori_loop(0, C1 // 16, dloop, 0, unroll=2)
        pltpu.async_copy(h_hbm.at[didx[b]], rows[b], semg[b])

    def scale(b):
        def eloop(e2, carry3):
            for u in range(4):
                e = e2 * 4 + u
                wv = plsc.load_gather(wq[b], [jnp.full((16,), e, jnp.int32)])
                wb16 = plsc.pack(wv, wv, format=plsc.PackFormat.INTERLEAVED)
                for k2 in range(W1 // 32):
                    rows[b][e, pl.ds(k2 * 32, 32)] = (
                        rows[b][e, pl.ds(k2 * 32, 32)] * wb16)
            return carry3
        lax.fori_loop(0, C1 // 4, eloop, 0, unroll=2)

    def per_head(i, carry):
        h = c * (HEADS // NC) + i

        def zr(e, carry2):
            for k in range(W1 // 32):
                rows0[e, pl.ds(k * 32, 32)] = jnp.zeros((32,), jnp.bfloat16)
            return carry2
        lax.fori_loop(0, C1, zr, 0)

        @pl.when(s < NS - 1)
        def _():
            pltpu.sync_copy(rows0, agg_sh.at[pl.ds(rbase, C1)])
            pltpu.sync_copy(rows0.at[pl.ds(0, SPAN - C1)],
                            agg_sh.at[pl.ds(rbase + C1, SPAN - C1)])

        @pl.when(s == NS - 1)
        def _():
            pltpu.sync_copy(rows0, agg_sh.at[pl.ds(15 * SPAN, TAIL)])
        plsc.subcore_barrier()

        for b in range(2):
            issue_wq(b, h, b)
            didx_gather(b, h, b)

        def pair_loop(k, carry2):
            for b in range(2):
                kk = 2 * k + b
                pltpu.make_async_copy(h_hbm.at[didx[b]], rows[b],
                                      semg[b]).wait()
                wait_wq(b)
                scale(b)
                pltpu.async_copy(rows[b], agg_sh.at[srcA.at[kk]], sems[b],
                                 add=True)

            @pl.when(k + 1 < NCH // 2)
            def _():
                for b in range(2):
                    kk2 = 2 * (k + 1) + b
                    pltpu.make_async_copy(rows[b], agg_sh.at[srcA.at[0]],
                                          sems[b]).wait()
                    issue_wq(kk2, h, b)
                    didx_gather(kk2, h, b)
            return carry2
        lax.fori_loop(0, NCH // 2, pair_loop, 0)
        for b in range(2):
            pltpu.make_async_copy(rows[b], agg_sh.at[srcA.at[0]],
                                  sems[b]).wait()
        plsc.subcore_barrier()

        @pl.when(s < NS - 1)
        def _():
            pltpu.sync_copy(agg_sh.at[pl.ds(rbase, SPAN)],
                            out_hbm.at[pl.ds(h * N + rbase, SPAN)])

        @pl.when(s == NS - 1)
        def _():
            pltpu.sync_copy(agg_sh.at[pl.ds(15 * SPAN, TAIL)],
                            out_hbm.at[pl.ds(h * N + 15 * SPAN, TAIL)])
        plsc.subcore_barrier()
        return carry
    lax.fori_loop(0, HEADS // NC, per_head, 0)


def _edges1(h_flat, w_flat, src2d, dst2d):
    return pl.kernel(
        _edges1_body,
        mesh=_MESH,
        compiler_params=_SC_PARAMS,
        out_type=jax.ShapeDtypeStruct((HEADS * N, W1), jnp.bfloat16),
        scratch_types=[
            pltpu.VMEM((ET1 // C1, C1), jnp.int32),
            pltpu.VMEM((ET1 // C1, C1), jnp.int32),
            pltpu.VMEM((C1,), jnp.float32),
            pltpu.VMEM((C1,), jnp.float32),
            pltpu.VMEM((C1,), jnp.int32),
            pltpu.VMEM((C1,), jnp.int32),
            pltpu.VMEM((C1, W1), jnp.bfloat16),
            pltpu.VMEM((C1, W1), jnp.bfloat16),
            pltpu.SemaphoreType.DMA,
            pltpu.SemaphoreType.DMA,
            pltpu.SemaphoreType.DMA,
            pltpu.SemaphoreType.DMA,
            pltpu.SemaphoreType.DMA,
            pltpu.SemaphoreType.DMA,
            pltpu.VMEM_SHARED((N, W1), jnp.bfloat16),
        ],
    )(h_flat, w_flat, src2d, dst2d)


def _dense2_body(agg_ref, wo_ref, ao_ref, h2_ref, fo_ref):
    parts = []
    for h in range(HEADS):
        a = agg_ref[h, :, 0:HID].astype(jnp.float32)
        rs = agg_ref[h, :, HID:HID + 1].astype(jnp.float32)
        parts.append(_elu(a / (rs + 1e-16)))
    h1 = jnp.concatenate(parts, axis=1)
    h2 = jnp.dot(h1, wo_ref[...], preferred_element_type=jnp.float32)
    h2_ref[:, 0:NCLS] = h2
    h2_ref[:, NCLS:NCLS + 1] = jnp.ones((BN, 1), jnp.float32)
    h2_ref[:, NCLS + 1:W2] = jnp.zeros((BN, W2 - NCLS - 1), jnp.float32)
    fo_ref[0, 0, :] = jnp.dot(h2, ao_ref[0, 0:NCLS])
    fo_ref[0, 1, :] = jnp.dot(h2, ao_ref[0, NCLS:2 * NCLS])


def _dense2(agg1, W_out, a_out2d):
    return pl.pallas_call(
        _dense2_body,
        grid=(N // BN,),
        in_specs=[
            pl.BlockSpec((HEADS, BN, W1), lambda i: (0, i, 0)),
            pl.BlockSpec((HEADS * HID, NCLS), lambda i: (0, 0)),
            pl.BlockSpec((1, 2 * NCLS), lambda i: (0, 0)),
        ],
        out_specs=[
            pl.BlockSpec((BN, W2), lambda i: (i, 0)),
            pl.BlockSpec((1, 2, BN), lambda i: (i, 0, 0)),
        ],
        out_shape=[
            jax.ShapeDtypeStruct((N, W2), jnp.float32),
            jax.ShapeDtypeStruct((NBLK, 2, BN), jnp.float32),
        ],
    )(agg1, W_out, a_out2d)


def _edges2_body(h_hbm, fo_hbm, src_hbm, dst_hbm, out_hbm,
                 f1_vm, f2_vm, srcA, dstA, rows0, rows1,
                 wbuf0, wbuf1, semg0, semg1, sems0, sems1, agg_sh):
    c = lax.axis_index("c")
    s = lax.axis_index("s")
    rbase = s * SPAN
    NCH = ET2 // C2
    rows = (rows0, rows1)
    wbuf = (wbuf0, wbuf1)
    semg = (semg0, semg1)
    sems = (sems0, sems1)

    wid = c * NS + s
    pltpu.sync_copy(src_hbm.at[pl.ds(wid * NCH, NCH)], srcA)
    pltpu.sync_copy(dst_hbm.at[pl.ds(wid * NCH, NCH)], dstA)
    for k in range(NBLK):
        pltpu.sync_copy(fo_hbm.at[pl.ds(k * 2 * BN, BN)],
                        f1_vm.at[pl.ds(k * BN, BN)])
        pltpu.sync_copy(fo_hbm.at[pl.ds((k * 2 + 1) * BN, BN)],
                        f2_vm.at[pl.ds(k * BN, BN)])

    def w_only(kk, b):
        def wloop(j, carry3):
            sv = srcA[kk, pl.ds(j * 16, 16)]
            dv = dstA[kk, pl.ds(j * 16, 16)]
            z = plsc.load_gather(f1_vm, [sv]) + plsc.load_gather(f2_vm, [dv])
            wbuf[b][pl.ds(j * 16, 16)] = jnp.exp(jnp.minimum(-z, -SLOPE * z))
            return carry3
        lax.fori_loop(0, C2 // 16, wloop, 0, unroll=2)

    def scale(b):
        def eloop(e2, carry3):
            for u in range(2):
                e = e2 * 2 + u
                wv = plsc.load_gather(wbuf[b], [jnp.full((16,), e, jnp.int32)])
                for k2 in range(W2 // 16):
                    rows[b][e, pl.ds(k2 * 16, 16)] = (
                        rows[b][e, pl.ds(k2 * 16, 16)] * wv)
            return carry3
        lax.fori_loop(0, C2 // 2, eloop, 0, unroll=2)

    def zr(e, carry):
        for k in range(W2 // 16):
            rows0[e, pl.ds(k * 16, 16)] = jnp.zeros((16,), jnp.float32)
        return carry
    lax.fori_loop(0, C2, zr, 0)

    @pl.when(s < NS - 1)
    def _():
        pltpu.sync_copy(rows0, agg_sh.at[pl.ds(rbase, C2)])
        pltpu.sync_copy(rows0.at[pl.ds(0, SPAN - C2)],
                        agg_sh.at[pl.ds(rbase + C2, SPAN - C2)])

    @pl.when(s == NS - 1)
    def _():
        pltpu.sync_copy(rows0, agg_sh.at[pl.ds(15 * SPAN, TAIL)])
    plsc.subcore_barrier()

    for b in range(2):
        w_only(b, b)
        pltpu.async_copy(h_hbm.at[dstA.at[b]], rows[b], semg[b])

    def pair_loop(k, carry2):
        for b in range(2):
            kk = 2 * k + b
            pltpu.make_async_copy(h_hbm.at[dstA.at[0]], rows[b],
                                  semg[b]).wait()
            scale(b)
            pltpu.async_copy(rows[b], agg_sh.at[srcA.at[kk]], sems[b],
                             add=True)

        @pl.when(k + 1 < NCH // 2)
        def _():
            for b in range(2):
                kk2 = 2 * (k + 1) + b
                w_only(kk2, b)
                pltpu.make_async_copy(rows[b], agg_sh.at[srcA.at[0]],
                                      sems[b]).wait()
                pltpu.async_copy(h_hbm.at[dstA.at[kk2]], rows[b], semg[b])
        return carry2
    lax.fori_loop(0, NCH // 2, pair_loop, 0)
    for b in range(2):
        pltpu.make_async_copy(rows[b], agg_sh.at[srcA.at[0]], sems[b]).wait()
    plsc.subcore_barrier()

    @pl.when(s < NS - 1)
    def _():
        pltpu.sync_copy(agg_sh.at[pl.ds(rbase, SPAN)],
                        out_hbm.at[pl.ds(c * N + rbase, SPAN)])

    @pl.when(s == NS - 1)
    def _():
        pltpu.sync_copy(agg_sh.at[pl.ds(15 * SPAN, TAIL)],
                        out_hbm.at[pl.ds(c * N + 15 * SPAN, TAIL)])


def _edges2(h2aug, fo_flat, src2d, dst2d):
    return pl.kernel(
        _edges2_body,
        mesh=_MESH,
        compiler_params=_SC_PARAMS,
        out_type=jax.ShapeDtypeStruct((NC * N, W2), jnp.float32),
        scratch_types=[
            pltpu.VMEM((N,), jnp.float32),
            pltpu.VMEM((N,), jnp.float32),
            pltpu.VMEM((ET2 // C2, C2), jnp.int32),
            pltpu.VMEM((ET2 // C2, C2), jnp.int32),
            pltpu.VMEM((C2, W2), jnp.float32),
            pltpu.VMEM((C2, W2), jnp.float32),
            pltpu.VMEM((C2,), jnp.float32),
            pltpu.VMEM((C2,), jnp.float32),
            pltpu.SemaphoreType.DMA,
            pltpu.SemaphoreType.DMA,
            pltpu.SemaphoreType.DMA,
            pltpu.SemaphoreType.DMA,
            pltpu.VMEM_SHARED((N, W2), jnp.float32),
        ],
    )(h2aug, fo_flat, src2d, dst2d)


def _final_body(p0_ref, p1_ref, o_ref):
    t = p0_ref[...] + p1_ref[...]
    agg = t[:, 0:NCLS]
    rs = t[:, NCLS:NCLS + 1]
    o = _elu(agg / (rs + 1e-16))
    o_ref[...] = jax.nn.log_softmax(o, axis=-1)


def _final(p):
    nblk = N // BN
    return pl.pallas_call(
        _final_body,
        grid=(nblk,),
        in_specs=[
            pl.BlockSpec((BN, W2), lambda i: (i, 0)),
            pl.BlockSpec((BN, W2), lambda i: (i + nblk, 0)),
        ],
        out_specs=pl.BlockSpec((BN, NCLS), lambda i: (i, 0)),
        out_shape=jax.ShapeDtypeStruct((N, NCLS), jnp.float32),
    )(p, p)


def kernel(x, adj, Wh, ah, W_out, a_out):
    src = adj[0]
    dst = adj[1]
    src2d = src.reshape(E // C1, C1)
    dst2d = dst.reshape(E // C1, C1)
    ff = _densef(x, Wh, ah)
    wall = _wpre(ff.reshape(NBLK * 2 * HEADS * BN), src2d, dst2d)
    h1aug = _dense1(x, Wh)
    agg1 = _edges1(h1aug.reshape(HEADS * N, W1), wall, src2d, dst2d)
    h2aug, fo = _dense2(agg1.reshape(HEADS, N, W1), W_out,
                        a_out.reshape(1, 2 * NCLS))
    p = _edges2(h2aug, fo.reshape(NBLK * 2 * BN), src2d, dst2d)
    return _final(p)
